# baseline (speedup 1.0000x reference)
import functools

import jax
import jax.numpy as jnp
from jax.experimental import pallas as pl
from jax.experimental.pallas import tpu as pltpu

N = 10000
E = 320000
H = 12
NEG = -1e30


def _t1_body(nf_ref, w_ref, oa_ref, ob_ref):
    y = jnp.dot(nf_ref[...], w_ref[...], preferred_element_type=jnp.float32)
    oa_ref[...] = y[:, :64]
    ob_ref[...] = y[:, 64:]


def _t2_body(g_ref, ef_ref, w1_ref, b1_ref, w2_ref, b2_ref, w3_ref, b3_ref,
             w4_ref, b4_ref, o1_ref, o2_ref):
    h = g_ref[...] + jnp.dot(ef_ref[...], w1_ref[...],
                             preferred_element_type=jnp.float32) + b1_ref[...]
    h = jnp.maximum(h, 0.0)
    h = jnp.maximum(jnp.dot(h, w2_ref[...], preferred_element_type=jnp.float32)
                    + b2_ref[...], 0.0)
    h = jnp.maximum(jnp.dot(h, w3_ref[...], preferred_element_type=jnp.float32)
                    + b3_ref[...], 0.0)
    y = jnp.dot(h, w4_ref[...], preferred_element_type=jnp.float32) + b4_ref[...]
    k = jax.nn.sigmoid(y[:, 28:29])
    lane = jax.lax.broadcasted_iota(jnp.int32, (y.shape[0], 16), 1)
    o1_ref[...] = jnp.where(lane < H, y[:, 0:16] * k, NEG)
    o2_ref[...] = jnp.where(lane < H, y[:, 16:32] * k, 0.0)


def _t3_body(nf_ref, m_ref, s_ref, w1n_ref, w1m_ref, w1s_ref, b1_ref,
             w2_ref, b2_ref, w3_ref, b3_ref, w4_ref, b4_ref, o_ref):
    m = m_ref[...]
    m = jnp.where(m < -1e29, 0.0, m)
    h = (jnp.dot(nf_ref[...], w1n_ref[...], preferred_element_type=jnp.float32)
         + jnp.dot(m, w1m_ref[...], preferred_element_type=jnp.float32)
         + jnp.dot(s_ref[...], w1s_ref[...], preferred_element_type=jnp.float32)
         + b1_ref[...])
    h = jnp.maximum(h, 0.0)
    h = jnp.maximum(jnp.dot(h, w2_ref[...], preferred_element_type=jnp.float32)
                    + b2_ref[...], 0.0)
    h = jnp.maximum(jnp.dot(h, w3_ref[...], preferred_element_type=jnp.float32)
                    + b3_ref[...], 0.0)
    o_ref[...] = jnp.dot(h, w4_ref[...], preferred_element_type=jnp.float32) + b4_ref[...]


def _full(shape):
    return pl.BlockSpec(shape, lambda i: (0,) * len(shape))


def kernel(nf, edge_index, ef, Wm1, bm1, Wm2, bm2, Wm3, bm3, Wm4, bm4,
           Wr1, br1, Wr2, br2, Wr3, br3, Wr4, br4):
    src = edge_index[0]
    dst = edge_index[1]

    Wab = jnp.concatenate([Wm1[:128], Wm1[128:256]], axis=1)
    BN = 2000
    A, B = pl.pallas_call(
        _t1_body,
        grid=(N // BN,),
        in_specs=[pl.BlockSpec((BN, 128), lambda i: (i, 0)), _full((128, 128))],
        out_specs=[pl.BlockSpec((BN, 64), lambda i: (i, 0)),
                   pl.BlockSpec((BN, 64), lambda i: (i, 0))],
        out_shape=[jax.ShapeDtypeStruct((N, 64), jnp.float32),
                   jax.ShapeDtypeStruct((N, 64), jnp.float32)],
    )(nf, Wab)

    G = A[src] + B[dst]

    W1e = Wm1[256:272]
    W4p = jnp.zeros((64, 32), jnp.float32)
    W4p = W4p.at[:, 0:12].set(Wm4[:, 1:13])
    W4p = W4p.at[:, 16:28].set(Wm4[:, 13:25])
    W4p = W4p.at[:, 28].set(Wm4[:, 0])
    b4p = jnp.zeros((32,), jnp.float32)
    b4p = b4p.at[0:12].set(bm4[1:13])
    b4p = b4p.at[16:28].set(bm4[13:25])
    b4p = b4p.at[28].set(bm4[0])

    BE = 2560
    EF1, EF2 = pl.pallas_call(
        _t2_body,
        grid=(E // BE,),
        in_specs=[pl.BlockSpec((BE, 64), lambda i: (i, 0)),
                  pl.BlockSpec((BE, 16), lambda i: (i, 0)),
                  _full((16, 64)), _full((1, 64)),
                  _full((64, 64)), _full((1, 64)),
                  _full((64, 64)), _full((1, 64)),
                  _full((64, 32)), _full((1, 32))],
        out_specs=[pl.BlockSpec((BE, 16), lambda i: (i, 0)),
                   pl.BlockSpec((BE, 16), lambda i: (i, 0))],
        out_shape=[jax.ShapeDtypeStruct((E, 16), jnp.float32),
                   jax.ShapeDtypeStruct((E, 16), jnp.float32)],
    )(G, ef, W1e, bm1.reshape(1, 64), Wm2, bm2.reshape(1, 64),
      Wm3, bm3.reshape(1, 64), W4p, b4p.reshape(1, 32))

    m = jax.ops.segment_max(EF1, dst, num_segments=N)
    s = jax.ops.segment_sum(EF2, dst, num_segments=N)

    W1n = Wr1[:128]
    W1m = jnp.zeros((16, 64), jnp.float32).at[:12].set(Wr1[128:140])
    W1s = jnp.zeros((16, 64), jnp.float32).at[:12].set(Wr1[140:152])
    out = pl.pallas_call(
        _t3_body,
        grid=(N // BN,),
        in_specs=[pl.BlockSpec((BN, 128), lambda i: (i, 0)),
                  pl.BlockSpec((BN, 16), lambda i: (i, 0)),
                  pl.BlockSpec((BN, 16), lambda i: (i, 0)),
                  _full((128, 64)), _full((16, 64)), _full((16, 64)),
                  _full((1, 64)),
                  _full((64, 64)), _full((1, 64)),
                  _full((64, 64)), _full((1, 64)),
                  _full((64, 128)), _full((1, 128))],
        out_specs=pl.BlockSpec((BN, 128), lambda i: (i, 0)),
        out_shape=jax.ShapeDtypeStruct((N, 128), jnp.float32),
    )(nf, m, s, W1n, W1m, W1s, br1.reshape(1, 64), Wr2, br2.reshape(1, 64),
      Wr3, br3.reshape(1, 64), Wr4, br4.reshape(1, 128))
    return out


# speedup vs baseline: 3.1257x; 3.1257x over previous
import functools

import jax
import jax.numpy as jnp
from jax import lax
from jax.experimental import pallas as pl
from jax.experimental.pallas import tpu as pltpu
from jax.experimental.pallas import tpu_sc as plsc

N = 10000
E = 320000
H = 12
NEG = -1e30

SC_NC = 2
SC_NS = 16
SC_NW = SC_NC * SC_NS
EH = E // 2

S1_PW = EH // SC_NW
S1_CH = 40
S1_NCH = S1_PW // S1_CH

S2_R = 5120
S2_PW = 10240
S2_CH = 320
S2_NP = 2
S2_CR = S2_R // SC_NS


def _s1_body(ab_hbm, src_hbm, dst_hbm, g_hbm,
             idxs, idxd, bs1, bd1, bs2, bd2, obuf, sg, so):
    wid = lax.axis_index("s") * SC_NC + lax.axis_index("c")
    base = wid * S1_PW
    pltpu.sync_copy(src_hbm.at[pl.ds(base, S1_PW)], idxs.at[pl.ds(0, S1_PW)])
    pltpu.sync_copy(src_hbm.at[pl.ds(EH + base, S1_PW)], idxs.at[pl.ds(S1_PW, S1_PW)])
    pltpu.sync_copy(dst_hbm.at[pl.ds(base, S1_PW)], idxd.at[pl.ds(0, S1_PW)])
    pltpu.sync_copy(dst_hbm.at[pl.ds(EH + base, S1_PW)], idxd.at[pl.ds(S1_PW, S1_PW)])

    def _gathers(j, slot):
        o = j * S1_CH
        return (
            pltpu.make_async_copy(ab_hbm.at[idxs.at[pl.ds(o, S1_CH)]],
                                  bs1.at[slot], sg.at[slot, 0]),
            pltpu.make_async_copy(ab_hbm.at[idxd.at[pl.ds(o, S1_CH)]],
                                  bd1.at[slot], sg.at[slot, 1]),
            pltpu.make_async_copy(ab_hbm.at[idxs.at[pl.ds(S1_PW + o, S1_CH)]],
                                  bs2.at[slot], sg.at[slot, 2]),
            pltpu.make_async_copy(ab_hbm.at[idxd.at[pl.ds(S1_PW + o, S1_CH)]],
                                  bd2.at[slot], sg.at[slot, 3]),
        )

    def _out_cp(j, slot):
        return pltpu.make_async_copy(
            obuf.at[slot], g_hbm.at[pl.ds(base + j * S1_CH, S1_CH)], so.at[slot])

    for b in range(3):
        for cp in _gathers(b, b):
            cp.start()

    def process(j, b, wait_prev_out):
        for cp in _gathers(j, b):
            cp.wait()
        if wait_prev_out is None:
            @pl.when(j >= 3)
            def _():
                _out_cp(j - 3, b).wait()
        elif wait_prev_out:
            _out_cp(j - 3, b).wait()

        def add_row(r, _):
            for c in range(4):
                sl = pl.ds(c * 16, 16)
                sr = pl.ds(64 + c * 16, 16)
                obuf[b, r, sl] = bs1[b, r, sl] + bd1[b, r, sr]
                obuf[b, r, sr] = bs2[b, r, sl] + bd2[b, r, sr]
            return 0
        lax.fori_loop(0, S1_CH, add_row, 0, unroll=2)

        _out_cp(j, b).start()

        @pl.when(j + 3 < S1_NCH)
        def _():
            for cp in _gathers(j + 3, b):
                cp.start()

    def body(g, _):
        for b in range(3):
            process(g * 3 + b, b, None)
        return 0

    lax.fori_loop(0, 41, body, 0)
    process(123, 0, True)
    process(124, 1, True)
    _out_cp(122, 2).wait()
    _out_cp(123, 0).wait()
    _out_cp(124, 1).wait()


def _s1_gather_add(ab, src, dst):
    mesh = plsc.VectorSubcoreMesh(core_axis_name="c", subcore_axis_name="s",
                                  num_cores=SC_NC, num_subcores=SC_NS)
    f = pl.kernel(
        _s1_body,
        out_type=jax.ShapeDtypeStruct((EH, 128), jnp.float32),
        mesh=mesh,
        scratch_types=[
            pltpu.VMEM((2 * S1_PW,), jnp.int32),
            pltpu.VMEM((2 * S1_PW,), jnp.int32),
            pltpu.VMEM((3, S1_CH, 128), jnp.float32),
            pltpu.VMEM((3, S1_CH, 128), jnp.float32),
            pltpu.VMEM((3, S1_CH, 128), jnp.float32),
            pltpu.VMEM((3, S1_CH, 128), jnp.float32),
            pltpu.VMEM((3, S1_CH, 128), jnp.float32),
            pltpu.SemaphoreType.DMA((3, 4)),
            pltpu.SemaphoreType.DMA((3,)),
        ],
    )
    return f(ab, src, dst)


def _s2_body(ef1_hbm, ef2_hbm, dst_hbm, omax_hbm, osum_hbm,
             idxd, tab, ebuf, sg):
    cid = lax.axis_index("c")
    sid = lax.axis_index("s")
    wid = sid * SC_NC + cid
    base = wid * S2_PW
    last = wid == SC_NW - 1
    nch = jnp.where(last, 8, 32)

    def ld_idx(c, _):
        pltpu.sync_copy(dst_hbm.at[pl.ds(base + c * 2560, 2560)],
                        idxd.at[pl.ds(c * 2560, 2560)])
        return 0
    lax.fori_loop(0, jnp.where(last, 1, 4), ld_idx, 0)

    for kind in ("sum", "max"):
        e_hbm = ef2_hbm if kind == "sum" else ef1_hbm
        o_hbm = osum_hbm if kind == "sum" else omax_hbm
        init = 0.0 if kind == "sum" else NEG
        op = jnp.add if kind == "sum" else jnp.maximum
        for p in range(S2_NP):
            lo = p * S2_R

            def init_row(r, _):
                tab[pl.ds(pl.multiple_of(r * 16, 16), 16)] = jnp.full(
                    (16,), init, jnp.float32)
                return 0
            lax.fori_loop(0, S2_R, init_row, 0)

            def chunk_cp(j, slot):
                off = pl.multiple_of((base + j * S2_CH) * 16, 128)
                return pltpu.make_async_copy(
                    e_hbm.at[pl.ds(off, S2_CH * 16)], ebuf.at[slot],
                    sg.at[slot])

            for b in range(2):
                chunk_cp(b, b).start()

            def scan_body(j, b):
                def scan_group(g, _):
                    goff = g * 16
                    dv = idxd[pl.ds(pl.multiple_of(j * S2_CH + goff, 16), 16)] - lo
                    for l in range(16):
                        d = dv[l]

                        @pl.when((d >= 0) & (d < S2_R))
                        def _():
                            t = pl.ds(pl.multiple_of(d * 16, 16), 16)
                            e = pl.ds(pl.multiple_of((goff + l) * 16, 16), 16)
                            tab[t] = op(tab[t], ebuf[b, e])
                    return 0
                lax.fori_loop(0, S2_CH // 16, scan_group, 0)

            def scan_chunk(g, _):
                for b in range(2):
                    j = g * 2 + b
                    chunk_cp(j, b).wait()
                    scan_body(j, b)

                    @pl.when(j + 2 < nch)
                    def _():
                        chunk_cp(j + 2, b).start()
                return 0
            lax.fori_loop(0, nch // 2, scan_chunk, 0)

            obase = wid * (S2_NP * S2_R * 16) + lo * 16
            pltpu.sync_copy(tab, o_hbm.at[pl.ds(pl.multiple_of(obase, 128),
                                                S2_R * 16)])


def _s2_segment(ef1, ef2, dst):
    mesh = plsc.VectorSubcoreMesh(core_axis_name="c", subcore_axis_name="s",
                                  num_cores=SC_NC, num_subcores=SC_NS)
    f = pl.kernel(
        _s2_body,
        out_type=[jax.ShapeDtypeStruct((SC_NW * S2_NP * S2_R * 16,), jnp.float32),
                  jax.ShapeDtypeStruct((SC_NW * S2_NP * S2_R * 16,), jnp.float32)],
        mesh=mesh,
        scratch_types=[
            pltpu.VMEM((S2_PW,), jnp.int32),
            pltpu.VMEM((S2_R * 16,), jnp.float32),
            pltpu.VMEM((2, S2_CH * 16), jnp.float32),
            pltpu.SemaphoreType.DMA((2,)),
        ],
    )
    om, os_ = f(ef1, ef2, dst)
    shp = (SC_NW, S2_NP * S2_R * 16 // 128, 128)
    return om.reshape(shp), os_.reshape(shp)


def _t25_body(m32_ref, s32_ref, m_ref, s_ref):
    m = m32_ref[0]
    s = s32_ref[0]
    for w in range(1, SC_NW):
        m = jnp.maximum(m, m32_ref[w])
        s = s + s32_ref[w]
    m_ref[...] = jnp.where(m < -1e29, 0.0, m)
    s_ref[...] = s


def _t1_body(nf_ref, w_ref, o_ref):
    o_ref[...] = jnp.dot(nf_ref[...], w_ref[...],
                         preferred_element_type=jnp.float32)


def _edge_mlp(g, ef, w1, b1, w2, b2, w3, b3, w4, b4):
    h = g + jnp.dot(ef, w1, preferred_element_type=jnp.float32) + b1
    h = jnp.maximum(h, 0.0)
    h = jnp.maximum(jnp.dot(h, w2, preferred_element_type=jnp.float32) + b2, 0.0)
    h = jnp.maximum(jnp.dot(h, w3, preferred_element_type=jnp.float32) + b3, 0.0)
    return jnp.dot(h, w4, preferred_element_type=jnp.float32) + b4


def _t2_body(g_ref, ef_ref, w1_ref, b1_ref, w2_ref, b2_ref, w3_ref, b3_ref,
             w4_ref, b4_ref, o1_ref, o2_ref):
    args = (w1_ref[...], b1_ref[...], w2_ref[...], b2_ref[...],
            w3_ref[...], b3_ref[...], w4_ref[...], b4_ref[...])
    lane = jax.lax.broadcasted_iota(jnp.int32, (g_ref.shape[0], 16), 1)
    for h in range(2):
        y = _edge_mlp(g_ref[:, 64 * h:64 * (h + 1)], ef_ref[h], *args)
        k = jax.nn.sigmoid(y[:, 28:29])
        o1_ref[h] = jnp.where(lane < H, y[:, 0:16] * k, NEG)
        o2_ref[h] = jnp.where(lane < H, y[:, 16:32] * k, 0.0)


def _t3_body(nf_ref, m_ref, s_ref,
             w1n_ref, w1m_ref, w1s_ref, b1_ref,
             w2_ref, b2_ref, w3_ref, b3_ref, w4_ref, b4_ref, o_ref):
    m = m_ref[...]
    s = s_ref[...]
    h = (jnp.dot(nf_ref[...], w1n_ref[...], preferred_element_type=jnp.float32)
         + jnp.dot(m, w1m_ref[...], preferred_element_type=jnp.float32)
         + jnp.dot(s, w1s_ref[...], preferred_element_type=jnp.float32)
         + b1_ref[...])
    h = jnp.maximum(h, 0.0)
    h = jnp.maximum(jnp.dot(h, w2_ref[...], preferred_element_type=jnp.float32)
                    + b2_ref[...], 0.0)
    h = jnp.maximum(jnp.dot(h, w3_ref[...], preferred_element_type=jnp.float32)
                    + b3_ref[...], 0.0)
    o_ref[...] = jnp.dot(h, w4_ref[...], preferred_element_type=jnp.float32) + b4_ref[...]


def _full(shape):
    return pl.BlockSpec(shape, lambda i: (0,) * len(shape))


def kernel(nf, edge_index, ef, Wm1, bm1, Wm2, bm2, Wm3, bm3, Wm4, bm4,
           Wr1, br1, Wr2, br2, Wr3, br3, Wr4, br4):
    src = edge_index[0].astype(jnp.int32)
    dst = edge_index[1].astype(jnp.int32)

    Wab = jnp.concatenate([Wm1[:128], Wm1[128:256]], axis=1)
    AB = pl.pallas_call(
        _t1_body,
        grid=(N // 2000,),
        in_specs=[pl.BlockSpec((2000, 128), lambda i: (i, 0)), _full((128, 128))],
        out_specs=pl.BlockSpec((2000, 128), lambda i: (i, 0)),
        out_shape=jax.ShapeDtypeStruct((N, 128), jnp.float32),
    )(nf, Wab)

    G2 = _s1_gather_add(AB, src, dst)

    W1e = Wm1[256:272]
    W4p = jnp.zeros((64, 32), jnp.float32)
    W4p = W4p.at[:, 0:12].set(Wm4[:, 1:13])
    W4p = W4p.at[:, 16:28].set(Wm4[:, 13:25])
    W4p = W4p.at[:, 28].set(Wm4[:, 0])
    b4p = jnp.zeros((32,), jnp.float32)
    b4p = b4p.at[0:12].set(bm4[1:13])
    b4p = b4p.at[16:28].set(bm4[13:25])
    b4p = b4p.at[28].set(bm4[0])

    BE = 1280
    ef2h = ef.reshape(2, EH, 16)
    EF1, EF2 = pl.pallas_call(
        _t2_body,
        grid=(EH // BE,),
        in_specs=[pl.BlockSpec((BE, 128), lambda i: (i, 0)),
                  pl.BlockSpec((2, BE, 16), lambda i: (0, i, 0)),
                  _full((16, 64)), _full((1, 64)),
                  _full((64, 64)), _full((1, 64)),
                  _full((64, 64)), _full((1, 64)),
                  _full((64, 32)), _full((1, 32))],
        out_specs=[pl.BlockSpec((2, BE, 16), lambda i: (0, i, 0)),
                   pl.BlockSpec((2, BE, 16), lambda i: (0, i, 0))],
        out_shape=[jax.ShapeDtypeStruct((2, EH, 16), jnp.float32),
                   jax.ShapeDtypeStruct((2, EH, 16), jnp.float32)],
    )(G2, ef2h, W1e, bm1.reshape(1, 64), Wm2, bm2.reshape(1, 64),
      Wm3, bm3.reshape(1, 64), W4p, b4p.reshape(1, 32))

    OM, OS = _s2_segment(EF1.reshape(E * 16), EF2.reshape(E * 16), dst)

    W1n = Wr1[:128]
    W1m = jnp.zeros((16, 64), jnp.float32).at[:12].set(Wr1[128:140])
    W1s = jnp.zeros((16, 64), jnp.float32).at[:12].set(Wr1[140:152])
    NP16 = S2_NP * S2_R * 16 // 128
    BP = NP16 // 8
    Mp, Sp = pl.pallas_call(
        _t25_body,
        grid=(8,),
        in_specs=[pl.BlockSpec((SC_NW, BP, 128), lambda i: (0, i, 0)),
                  pl.BlockSpec((SC_NW, BP, 128), lambda i: (0, i, 0))],
        out_specs=[pl.BlockSpec((BP, 128), lambda i: (i, 0)),
                   pl.BlockSpec((BP, 128), lambda i: (i, 0))],
        out_shape=[jax.ShapeDtypeStruct((NP16, 128), jnp.float32),
                   jax.ShapeDtypeStruct((NP16, 128), jnp.float32)],
    )(OM, OS)
    Mn = Mp.reshape(S2_NP * S2_R, 16)
    Sn = Sp.reshape(S2_NP * S2_R, 16)

    nfp = jnp.pad(nf, ((0, 240), (0, 0)))
    BN = 2560
    part = pl.BlockSpec((BN, 16), lambda i: (i, 0))
    outp = pl.pallas_call(
        _t3_body,
        grid=((N + 240) // BN,),
        in_specs=[pl.BlockSpec((BN, 128), lambda i: (i, 0)),
                  part, part,
                  _full((128, 64)), _full((16, 64)), _full((16, 64)),
                  _full((1, 64)),
                  _full((64, 64)), _full((1, 64)),
                  _full((64, 64)), _full((1, 64)),
                  _full((64, 128)), _full((1, 128))],
        out_specs=pl.BlockSpec((BN, 128), lambda i: (i, 0)),
        out_shape=jax.ShapeDtypeStruct((N + 240, 128), jnp.float32),
    )(nfp, Mn, Sn, W1n, W1m, W1s, br1.reshape(1, 64), Wr2,
      br2.reshape(1, 64), Wr3, br3.reshape(1, 64), Wr4, br4.reshape(1, 128))
    return outp[:N]
